# baseline (speedup 1.0000x reference)
import functools

import jax
import jax.numpy as jnp
from jax import lax
from jax.experimental import pallas as pl
from jax.experimental.pallas import tpu as pltpu
from jax.experimental.pallas import tpu_sc as plsc

_DIM = 128
_W = 256
_HID = 384
_N = 10000
_E = 320000
_EPS = 1e-05
_NC = 2
_NS = 16
_K = 128
_EPT = 10240
_EPAD = _NC * _NS * _EPT
_SUP = 16
_NSUP = _EPT // (_SUP * _K)
_CPT = _EPT // _K
_NPH = 4
_PH = 2560
_PHA = 2568
_ZPT = _PH // _NS
_ZST = 16
_CAP = _EPT + _K
_BN = 1000


def _dot_t(x, w):
    return lax.dot_general(x, w, (((1,), (1,)), ((), ())),
                           preferred_element_type=jnp.float32)


def _ffn1_body(v_ref, n1_ref, w1_ref, w2_ref, w3_ref, tbl_ref):
    x = v_ref[...]
    xn = x * lax.rsqrt(jnp.mean(x * x, axis=-1, keepdims=True) + _EPS) * n1_ref[...]
    a = _dot_t(xn, w1_ref[...])
    b = _dot_t(xn, w3_ref[...])
    h = a * jax.nn.sigmoid(a) * b
    va = jnp.tanh(_dot_t(h, w2_ref[...]) + x)
    lg = jnp.log(jnp.abs(va))
    ng = (va < 0).astype(jnp.float32)
    tbl_ref[:, 0, :] = lg
    tbl_ref[:, 1, :] = ng


def _ffn1_call(v, n1, w1, w2, w3):
    full = lambda shape: pl.BlockSpec(shape, lambda i: (0,) * len(shape))
    return pl.pallas_call(
        _ffn1_body,
        grid=(_N // _BN,),
        in_specs=[
            pl.BlockSpec((_BN, _DIM), lambda i: (i, 0)),
            full((1, _DIM)),
            full((_HID, _DIM)),
            full((_DIM, _HID)),
            full((_HID, _DIM)),
        ],
        out_specs=pl.BlockSpec((_BN, 2, _DIM), lambda i: (i, 0, 0)),
        out_shape=jax.ShapeDtypeStruct((_N, 2, _DIM), jnp.float32),
    )(v, n1, w1, w2, w3)


def _sc_body(tbl, i0f, i1f, out, idx0_v, idx1_v, idx0_w, idx1_w, row_v,
             row_w, stage_v, cil, ld0, ld1, acc_sh, sem, sem2):
    c = lax.axis_index("c")
    s = lax.axis_index("s")
    ebase = (c * _NS + s) * _EPT

    def _z(i, carry):
        for h in range(2):
            for j in range(_DIM // 16):
                stage_v[i, h, pl.ds(j * 16, 16)] = jnp.zeros((16,), jnp.float32)
        return carry
    lax.fori_loop(0, _ZST, _z, 0)

    for r in range(_NPH):
        base = r * _PH
        for t in range(_ZPT // _ZST):
            pltpu.sync_copy(stage_v,
                            acc_sh.at[pl.ds(s * _ZPT + t * _ZST, _ZST)])
        plsc.subcore_barrier()

        def _super(g, n):
            off = ebase + g * (_SUP * _K)
            pltpu.sync_copy(i0f.at[pl.ds(off, _SUP * _K)], ld0)
            pltpu.sync_copy(i1f.at[pl.ds(off, _SUP * _K)], ld1)

            def _sub(u, nn):
                for j in range(_K // 16):
                    sl = pl.ds(u * _K + j * 16, 16)
                    a0 = ld0[sl]
                    rel = ld1[sl] - base
                    ok = (rel >= 0) & (rel < _PH)
                    relc = jnp.where(ok, rel, _PH)
                    packed = a0 * 4096 + relc
                    _, sp = plsc.sort_key_val(ok.astype(jnp.int32), packed,
                                              descending=True)
                    cil[pl.ds(nn, 16)] = sp
                    nn = nn + plsc.all_reduce_population_count(ok)[0]
                return nn
            return lax.fori_loop(0, _SUP, _sub, n)
        cnt = lax.fori_loop(0, _NSUP, _super, jnp.int32(0))

        for j in range(_K // 16):
            cil[pl.ds(cnt + j * 16, 16)] = jnp.full((16,), _PH, jnp.int32)

        bufs = ((idx0_v, idx1_v, row_v, sem), (idx0_w, idx1_w, row_w, sem2))

        def _unpack_issue(k, b):
            i0b, i1b, rowb, semb = bufs[b]
            for j in range(_K // 16):
                sj = pl.ds(j * 16, 16)
                v = cil[pl.ds(k * _K + j * 16, 16)]
                i0b[sj] = lax.shift_right_logical(v, 12)
                i1b[sj] = lax.bitwise_and(v, 4095)
            pltpu.async_copy(tbl.at[i0b], rowb, semb)

        def _drain_scatter(b):
            i0b, i1b, rowb, semb = bufs[b]
            pltpu.make_async_copy(tbl.at[i0b], rowb, semb).wait()
            pltpu.sync_copy(rowb, acc_sh.at[i1b], add=True)

        @pl.when(0 < cnt)
        def _():
            _unpack_issue(0, 0)

        def _fire2(k2, carry):
            for b in range(2):
                k = k2 * 2 + b

                @pl.when(k * _K < cnt)
                def _():
                    @pl.when((k + 1) * _K < cnt)
                    def _():
                        _unpack_issue(k + 1, 1 - b)
                    _drain_scatter(b)
            return carry
        lax.fori_loop(0, (_CPT + 3) // 2, _fire2, 0)
        plsc.subcore_barrier()

        if r < _NPH - 1:
            pltpu.sync_copy(acc_sh.at[pl.ds(s * _ZPT, _ZPT)],
                            out.at[c, pl.ds(base + s * _ZPT, _ZPT)])
        else:
            @pl.when(s < 14)
            def _():
                pltpu.sync_copy(acc_sh.at[pl.ds(s * _ZPT, _ZPT)],
                                out.at[c, pl.ds(base + s * _ZPT, _ZPT)])

            @pl.when(s == 14)
            def _():
                pltpu.sync_copy(acc_sh.at[pl.ds(14 * _ZPT, 80)],
                                out.at[c, pl.ds(base + 14 * _ZPT, 80)])
        plsc.subcore_barrier()


def _sc_scatter(tbl, i0f, i1f):
    mesh = plsc.VectorSubcoreMesh(core_axis_name="c", subcore_axis_name="s")
    kern = functools.partial(
        pl.kernel,
        out_type=jax.ShapeDtypeStruct((_NC, _N, 2, _DIM), jnp.float32),
        mesh=mesh,
        compiler_params=pltpu.CompilerParams(needs_layout_passes=False),
        scratch_types=[
            pltpu.VMEM((_K,), jnp.int32),
            pltpu.VMEM((_K,), jnp.int32),
            pltpu.VMEM((_K,), jnp.int32),
            pltpu.VMEM((_K,), jnp.int32),
            pltpu.VMEM((_K, 2, _DIM), jnp.float32),
            pltpu.VMEM((_K, 2, _DIM), jnp.float32),
            pltpu.VMEM((_ZST, 2, _DIM), jnp.float32),
            pltpu.VMEM((_CAP,), jnp.int32),
            pltpu.VMEM((_SUP * _K,), jnp.int32),
            pltpu.VMEM((_SUP * _K,), jnp.int32),
            pltpu.VMEM_SHARED((_PHA, 2, _DIM), jnp.float32),
            pltpu.SemaphoreType.DMA,
            pltpu.SemaphoreType.DMA,
        ],
    )(_sc_body)
    return kern(tbl, i0f, i1f)


def _ffn2_body(acc_ref, n2_ref, w1_ref, w2_ref, w3_ref, ow_ref, ob_ref, out_ref):
    a = acc_ref[0] + acc_ref[1]
    accl = a[:, 0, :]
    accn = a[:, 1, :]
    sgn = 1.0 - 2.0 * (accn - 2.0 * jnp.floor(accn * 0.5))
    l = sgn * jnp.exp(accl)
    ln = l * lax.rsqrt(jnp.mean(l * l, axis=-1, keepdims=True) + _EPS) * n2_ref[...]
    a1 = _dot_t(ln, w1_ref[...])
    b1 = _dot_t(ln, w3_ref[...])
    h = a1 * jax.nn.sigmoid(a1) * b1
    l2 = _dot_t(h, w2_ref[...]) + l
    out_ref[...] = jnp.sum(l2 * ow_ref[...], axis=-1, keepdims=True) + ob_ref[...]


def _ffn2_call(acc, n2, w1, w2, w3, ow, ob):
    full = lambda shape: pl.BlockSpec(shape, lambda i: (0,) * len(shape))
    return pl.pallas_call(
        _ffn2_body,
        grid=(_N // _BN,),
        in_specs=[
            pl.BlockSpec((2, _BN, 2, _DIM), lambda i: (0, i, 0, 0)),
            full((1, _DIM)),
            full((_HID, _DIM)),
            full((_DIM, _HID)),
            full((_HID, _DIM)),
            full((1, _DIM)),
            full((1, 1)),
        ],
        out_specs=pl.BlockSpec((_BN, 1), lambda i: (i, 0)),
        out_shape=jax.ShapeDtypeStruct((_N, 1), jnp.float32),
    )(acc, n2, w1, w2, w3, ow, ob)


def kernel(v, data_to_logical, norm1_w, w1_a, w2_a, w3_a,
           norm2_w, w1_b, w2_b, w3_b, out_w, out_b):
    tbl = _ffn1_call(v, norm1_w.reshape(1, _DIM), w1_a, w2_a, w3_a)
    i0 = data_to_logical[0]
    i1 = data_to_logical[1]
    i0f = jnp.concatenate([i0, jnp.zeros((_EPAD - _E,), jnp.int32)])
    i1f = jnp.concatenate([i1, jnp.full((_EPAD - _E,), _N, jnp.int32)])
    acc = _sc_scatter(tbl, i0f, i1f)
    out2 = _ffn2_call(acc, norm2_w.reshape(1, _DIM),
                      w1_b, w2_b, w3_b, out_w, out_b.reshape(1, 1))
    return out2.reshape(_N)


# speedup vs baseline: 1.4690x; 1.4690x over previous
import functools

import jax
import jax.numpy as jnp
from jax import lax
from jax.experimental import pallas as pl
from jax.experimental.pallas import tpu as pltpu
from jax.experimental.pallas import tpu_sc as plsc

_DIM = 128
_HID = 384
_N = 10000
_E = 320000
_EPS = 1e-05
_HALF = 64
_NC = 2
_NS = 16
_K = 128
_NCHUNK = 2560
_EPAD = _NCHUNK * _K
_EPT = _EPAD // _NS
_SUP = 16
_NSUP = _EPT // (_SUP * _K)
_PH = 5120
_PHA = 5128
_ZPT = _PH // _NS
_ZST = 64
_S1 = 304
_CAP = 20608
_CAPT = _CAP
_BN = 1000


def _dot_t(x, w):
    return lax.dot_general(x, w, (((1,), (1,)), ((), ())),
                           preferred_element_type=jnp.float32)


def _ffn1_body(v_ref, n1_ref, w1_ref, w2_ref, w3_ref, tbl_ref):
    x = v_ref[...]
    xn = x * lax.rsqrt(jnp.mean(x * x, axis=-1, keepdims=True) + _EPS) * n1_ref[...]
    a = _dot_t(xn, w1_ref[...])
    b = _dot_t(xn, w3_ref[...])
    h = a * jax.nn.sigmoid(a) * b
    va = jnp.tanh(_dot_t(h, w2_ref[...]) + x)
    lg = jnp.log(jnp.abs(va))
    ng = (va < 0).astype(jnp.float32)
    tbl_ref[0] = jnp.concatenate([lg[:, :_HALF], ng[:, :_HALF]], axis=-1)
    tbl_ref[1] = jnp.concatenate([lg[:, _HALF:], ng[:, _HALF:]], axis=-1)


def _ffn1_call(v, n1, w1, w2, w3):
    full = lambda shape: pl.BlockSpec(shape, lambda i: (0,) * len(shape))
    return pl.pallas_call(
        _ffn1_body,
        grid=(_N // _BN,),
        in_specs=[
            pl.BlockSpec((_BN, _DIM), lambda i: (i, 0)),
            full((1, _DIM)),
            full((_HID, _DIM)),
            full((_DIM, _HID)),
            full((_HID, _DIM)),
        ],
        out_specs=pl.BlockSpec((2, _BN, _DIM), lambda i: (0, i, 0)),
        out_shape=jax.ShapeDtypeStruct((2, _N, _DIM), jnp.float32),
    )(v, n1, w1, w2, w3)


def _sc_body(tbl, i0f, i1f, out, idx0_v, idx1_v, idx0_w, idx1_w, idx0_x,
             idx1_x, row_v, row_w, row_x, stage_v, cil, cnb, ld0, ld1, acc_sh,
             sem, sem2, sem3):
    c = lax.axis_index("c")
    s = lax.axis_index("s")
    coff = c * _N
    cpt = _NCHUNK // _NS

    def _z(i, carry):
        for j in range(_DIM // 16):
            stage_v[i, pl.ds(j * 16, 16)] = jnp.zeros((16,), jnp.float32)
        return carry
    lax.fori_loop(0, _ZST, _z, 0)

    for r in range(2):
        base = r * _PH
        for t in range(_ZPT // _ZST):
            pltpu.sync_copy(stage_v,
                            acc_sh.at[pl.ds(s * _ZPT + t * _ZST, _ZST)])
        plsc.subcore_barrier()

        def _super(g, n):
            off = s * _EPT + g * (_SUP * _K)
            pltpu.sync_copy(i0f.at[pl.ds(off, _SUP * _K)], ld0)
            pltpu.sync_copy(i1f.at[pl.ds(off, _SUP * _K)], ld1)

            def _sub(u, nn):
                for j in range(_K // 16):
                    sl = pl.ds(u * _K + j * 16, 16)
                    a0 = ld0[sl] + coff
                    rel = ld1[sl] - base
                    ok = (rel >= 0) & (rel < _PH)
                    relc = jnp.where(ok, rel, _PH)
                    packed = a0 * 8192 + relc
                    _, sp = plsc.sort_key_val(ok.astype(jnp.int32), packed,
                                              descending=True)
                    cil[pl.ds(nn, 16)] = sp
                    nn = nn + plsc.all_reduce_population_count(ok)[0]
                return nn
            return lax.fori_loop(0, _SUP, _sub, n)
        cnt = lax.fori_loop(0, _NSUP, _super, jnp.int32(0))

        jp = coff * 8192 + _PH
        for j in range(_K // 16):
            cil[pl.ds(cnt + j * 16, 16)] = jnp.zeros((16,), jnp.int32) + jp

        bufs = ((idx0_v, idx1_v, row_v, sem), (idx0_w, idx1_w, row_w, sem2),
                (idx0_x, idx1_x, row_x, sem3))

        def _unpack_issue(k, b):
            i0b, i1b, rowb, semb = bufs[b]
            for j in range(_K // 16):
                sj = pl.ds(j * 16, 16)
                v = cil[pl.ds(k * _K + j * 16, 16)]
                i0b[sj] = lax.shift_right_logical(v, 13)
                i1b[sj] = lax.bitwise_and(v, 8191)
            pltpu.async_copy(tbl.at[i0b], rowb, semb)

        def _drain_scatter(b):
            i0b, i1b, rowb, semb = bufs[b]
            pltpu.make_async_copy(tbl.at[i0b], rowb, semb).wait()
            pltpu.sync_copy(rowb, acc_sh.at[i1b], add=True)

        @pl.when(0 < cnt)
        def _():
            _unpack_issue(0, 0)

        @pl.when(_K < cnt)
        def _():
            _unpack_issue(1, 1)

        def _fire3(k3, carry):
            for b in range(3):
                k = k3 * 3 + b

                @pl.when(k * _K < cnt)
                def _():
                    @pl.when((k + 2) * _K < cnt)
                    def _():
                        _unpack_issue(k + 2, (b + 2) % 3)
                    _drain_scatter(b)
            return carry
        lax.fori_loop(0, (cpt + 5) // 3, _fire3, 0)
        plsc.subcore_barrier()

        if r == 0:
            pltpu.sync_copy(acc_sh.at[pl.ds(s * _ZPT, _ZPT)],
                            out.at[c, pl.ds(s * _ZPT, _ZPT)])
        else:
            @pl.when(s < _NS - 1)
            def _():
                pltpu.sync_copy(acc_sh.at[pl.ds(s * _S1, _S1)],
                                out.at[c, pl.ds(_PH + s * _S1, _S1)])

            @pl.when(s == _NS - 1)
            def _():
                pltpu.sync_copy(acc_sh.at[pl.ds((_NS - 1) * _S1, _ZPT)],
                                out.at[c, pl.ds(_PH + (_NS - 1) * _S1, _ZPT)])
        plsc.subcore_barrier()


def _sc_scatter(tbl, i0r, i1r):
    mesh = plsc.VectorSubcoreMesh(core_axis_name="c", subcore_axis_name="s")
    kern = functools.partial(
        pl.kernel,
        out_type=jax.ShapeDtypeStruct((_NC, _N, _DIM), jnp.float32),
        mesh=mesh,
        compiler_params=pltpu.CompilerParams(needs_layout_passes=False),
        scratch_types=[
            pltpu.VMEM((_K,), jnp.int32),
            pltpu.VMEM((_K,), jnp.int32),
            pltpu.VMEM((_K,), jnp.int32),
            pltpu.VMEM((_K,), jnp.int32),
            pltpu.VMEM((_K,), jnp.int32),
            pltpu.VMEM((_K,), jnp.int32),
            pltpu.VMEM((_K, _DIM), jnp.float32),
            pltpu.VMEM((_K, _DIM), jnp.float32),
            pltpu.VMEM((_K, _DIM), jnp.float32),
            pltpu.VMEM((_ZST, _DIM), jnp.float32),
            pltpu.VMEM((_CAPT,), jnp.int32),
            pltpu.VMEM((16,), jnp.int32),
            pltpu.VMEM((_SUP * _K,), jnp.int32),
            pltpu.VMEM((_SUP * _K,), jnp.int32),
            pltpu.VMEM_SHARED((_PHA, _DIM), jnp.float32),
            pltpu.SemaphoreType.DMA,
            pltpu.SemaphoreType.DMA,
            pltpu.SemaphoreType.DMA,
        ],
    )(_sc_body)
    return kern(tbl, i0r, i1r)


def _ffn2_body(acc_ref, n2_ref, w1_ref, w2_ref, w3_ref, ow_ref, ob_ref, out_ref):
    acc = acc_ref[...]
    accl = jnp.concatenate([acc[0, :, :_HALF], acc[1, :, :_HALF]], axis=-1)
    accn = jnp.concatenate([acc[0, :, _HALF:], acc[1, :, _HALF:]], axis=-1)
    sgn = 1.0 - 2.0 * (accn - 2.0 * jnp.floor(accn * 0.5))
    l = sgn * jnp.exp(accl)
    ln = l * lax.rsqrt(jnp.mean(l * l, axis=-1, keepdims=True) + _EPS) * n2_ref[...]
    a = _dot_t(ln, w1_ref[...])
    b = _dot_t(ln, w3_ref[...])
    h = a * jax.nn.sigmoid(a) * b
    l2 = _dot_t(h, w2_ref[...]) + l
    out_ref[...] = jnp.sum(l2 * ow_ref[...], axis=-1, keepdims=True) + ob_ref[...]


def _ffn2_call(acc, n2, w1, w2, w3, ow, ob):
    full = lambda shape: pl.BlockSpec(shape, lambda i: (0,) * len(shape))
    return pl.pallas_call(
        _ffn2_body,
        grid=(_N // _BN,),
        in_specs=[
            pl.BlockSpec((2, _BN, _DIM), lambda i: (0, i, 0)),
            full((1, _DIM)),
            full((_HID, _DIM)),
            full((_DIM, _HID)),
            full((_HID, _DIM)),
            full((1, _DIM)),
            full((1, 1)),
        ],
        out_specs=pl.BlockSpec((_BN, 1), lambda i: (i, 0)),
        out_shape=jax.ShapeDtypeStruct((_N, 1), jnp.float32),
    )(acc, n2, w1, w2, w3, ow, ob)


def kernel(v, data_to_logical, norm1_w, w1_a, w2_a, w3_a,
           norm2_w, w1_b, w2_b, w3_b, out_w, out_b):
    tbl = _ffn1_call(v, norm1_w.reshape(1, _DIM), w1_a, w2_a, w3_a)
    i0 = data_to_logical[0]
    i1 = data_to_logical[1]
    i0f = jnp.concatenate([i0, jnp.zeros((_EPAD - _E,), jnp.int32)])
    i1f = jnp.concatenate([i1, jnp.full((_EPAD - _E,), _N, jnp.int32)])
    acc = _sc_scatter(tbl.reshape(2 * _N, _DIM), i0f, i1f)
    out2 = _ffn2_call(acc, norm2_w.reshape(1, _DIM),
                      w1_b, w2_b, w3_b, out_w, out_b.reshape(1, 1))
    return out2.reshape(_N)
